# baseline (speedup 1.0000x reference)
import functools

import jax
import jax.numpy as jnp
from jax import lax
from jax.experimental import pallas as pl
from jax.experimental.pallas import tpu as pltpu
from jax.experimental.pallas import tpu_sc as plsc

DIM = 32
B = 16384
EPS = 1e-5

NC = 2
NS = 16
NW = NC * NS
BPW = B // NW
CH = 128
NCH = BPW // CH

_mesh = plsc.VectorSubcoreMesh(core_axis_name="c", subcore_axis_name="s")


@functools.partial(
    pl.kernel,
    mesh=_mesh,
    compiler_params=pltpu.CompilerParams(use_tc_tiling_on_sc=False),
    out_type=jax.ShapeDtypeStruct((NW, NCH, CH, DIM), jnp.float32),
    scratch_types=[
        pltpu.VMEM((NCH, CH), jnp.int32),
        pltpu.VMEM((NCH, CH, DIM), jnp.float32),
        pltpu.SemaphoreType.DMA,
    ],
)
def _sc_gather(idx_hbm, table_hbm, out_hbm, idx_v, rows_v, sem):
    wid = lax.axis_index("s") * NC + lax.axis_index("c")
    pltpu.sync_copy(idx_hbm.at[wid], idx_v)
    copies = [
        pltpu.async_copy(table_hbm.at[idx_v.at[k]], rows_v.at[k], sem)
        for k in range(NCH)
    ]
    for c in copies:
        c.wait()
    pltpu.sync_copy(rows_v, out_hbm.at[wid])


def _tc_bn_elu(x_ref, g_ref, b_ref, o_ref):
    x = x_ref[...]
    s = jnp.sum(x, axis=0, keepdims=True)
    q = jnp.sum(x * x, axis=0, keepdims=True)
    s32 = s[:, 0:32] + s[:, 32:64] + s[:, 64:96] + s[:, 96:128]
    q32 = q[:, 0:32] + q[:, 32:64] + q[:, 64:96] + q[:, 96:128]
    mean = s32 * (1.0 / B)
    var = q32 * (1.0 / B) - mean * mean
    inv = lax.rsqrt(var + EPS)
    scale32 = g_ref[...] * inv
    shift32 = b_ref[...] - mean * scale32
    scale = jnp.concatenate([scale32] * 4, axis=1)
    shift = jnp.concatenate([shift32] * 4, axis=1)
    y = x * scale + shift
    o_ref[...] = jnp.where(y > 0, y, jnp.exp(y) - 1.0)


def kernel(x, table, gamma, beta):
    idx = x.astype(jnp.int32).reshape(NW, NCH, CH)
    emb = _sc_gather(idx, table)
    emb2 = emb.reshape(B // 4, 4 * DIM)
    out2 = pl.pallas_call(
        _tc_bn_elu,
        out_shape=jax.ShapeDtypeStruct((B // 4, 4 * DIM), jnp.float32),
    )(emb2, gamma.reshape(1, DIM), beta.reshape(1, DIM))
    return out2.reshape(B, DIM)


# speedup vs baseline: 1.4970x; 1.4970x over previous
import functools

import jax
import jax.numpy as jnp
from jax import lax
from jax.experimental import pallas as pl
from jax.experimental.pallas import tpu as pltpu
from jax.experimental.pallas import tpu_sc as plsc

DIM = 32
B = 16384
EPS = 1e-5

NC = 2
NS = 16
NW = NC * NS
BPW = B // NW
K = 16

_mesh = plsc.VectorSubcoreMesh(core_axis_name="c", subcore_axis_name="s")


@functools.partial(
    pl.kernel,
    mesh=_mesh,
    out_type=jax.ShapeDtypeStruct((B, DIM), jnp.float32),
    scratch_types=[
        pltpu.VMEM((BPW,), jnp.int32),
        pltpu.VMEM((BPW, DIM), jnp.float32),
        pltpu.SemaphoreType.DMA,
    ],
)
def _sc_gather(idx_hbm, table_hbm, out_hbm, idx_v, rows_v, sem):
    wid = lax.axis_index("s") * NC + lax.axis_index("c")
    base = wid * BPW
    pltpu.sync_copy(idx_hbm.at[pl.ds(base, BPW)], idx_v)

    def chunk(ci, carry):
        off = ci * K
        vec = idx_v[pl.ds(off, K)]
        copies = []
        for j in range(K):
            r = vec[j]
            copies.append(
                pltpu.async_copy(
                    table_hbm.at[pl.ds(r, 1)],
                    rows_v.at[pl.ds(off + j, 1)],
                    sem,
                )
            )
        for c in copies:
            c.wait()
        return carry

    lax.fori_loop(0, BPW // K, chunk, 0)
    pltpu.sync_copy(rows_v, out_hbm.at[pl.ds(base, BPW)])


CHUNK = 512


def _tc_bn_elu(x_ref, g_ref, b_ref, o_ref):
    R = B // 4

    def stat_body(i, carry):
        s, q = carry
        xb = x_ref[pl.ds(i * CHUNK, CHUNK), :]
        s = s + jnp.sum(xb, axis=0, keepdims=True)
        q = q + jnp.sum(xb * xb, axis=0, keepdims=True)
        return s, q

    zero = jnp.zeros((1, 4 * DIM), jnp.float32)
    s, q = lax.fori_loop(0, R // CHUNK, stat_body, (zero, zero))
    s32 = s[:, 0:32] + s[:, 32:64] + s[:, 64:96] + s[:, 96:128]
    q32 = q[:, 0:32] + q[:, 32:64] + q[:, 64:96] + q[:, 96:128]
    mean = s32 * (1.0 / B)
    var = q32 * (1.0 / B) - mean * mean
    inv = lax.rsqrt(var + EPS)
    scale32 = g_ref[...] * inv
    shift32 = b_ref[...] - mean * scale32
    scale = jnp.concatenate([scale32] * 4, axis=1)
    shift = jnp.concatenate([shift32] * 4, axis=1)

    def out_body(i, carry):
        xb = x_ref[pl.ds(i * CHUNK, CHUNK), :]
        y = xb * scale + shift
        o_ref[pl.ds(i * CHUNK, CHUNK), :] = jnp.where(y > 0, y, jnp.exp(y) - 1.0)
        return carry

    lax.fori_loop(0, R // CHUNK, out_body, 0)


def kernel(x, table, gamma, beta):
    idx = x.astype(jnp.int32)
    emb = _sc_gather(idx, table)
    emb2 = emb.reshape(B // 4, 4 * DIM)
    out2 = pl.pallas_call(
        _tc_bn_elu,
        out_shape=jax.ShapeDtypeStruct((B // 4, 4 * DIM), jnp.float32),
    )(emb2, gamma.reshape(1, DIM), beta.reshape(1, DIM))
    return out2.reshape(B, DIM)
